# baseline (speedup 1.0000x reference)
import functools

import jax
import jax.numpy as jnp
from jax import lax
from jax.experimental import pallas as pl
from jax.experimental.pallas import tpu as pltpu
from jax.experimental.pallas import tpu_sc as plsc

NC = 2
NS = 16
NW = NC * NS
SUB = 128
CH = 2 * SUB
F32 = jnp.float32


def _ceil_to(v, m):
    return -(-v // m) * m


def _mesh():
    return plsc.VectorSubcoreMesh(core_axis_name="c", subcore_axis_name="s")


SEG_G = 20


def _segsum(table, src4, dst4, n_nodes_pad):
    n_groups = src4.shape[0]
    rows_per_tile = n_nodes_pad // NS

    def body(table_h, src_h, dst_h, out_h, sidx, didx, rows, accum, semg, sems):
        cid = lax.axis_index("c")
        sid = lax.axis_index("s")
        wid = sid * NC + cid

        def zrow(i, carry):
            for j in range(8):
                rows[0, i, pl.ds(16 * j, 16)] = jnp.zeros((16,), F32)
            return carry

        lax.fori_loop(0, SUB, zrow, 0)
        base = sid * rows_per_tile
        off = 0
        while off < rows_per_tile:
            n = min(SUB, rows_per_tile - off)
            pltpu.sync_copy(rows.at[0, pl.ds(0, n)], accum.at[pl.ds(base + off, n)])
            off += n
        plsc.subcore_barrier()

        def group(g, carry):
            pltpu.sync_copy(src_h.at[g, wid], sidx)
            pltpu.sync_copy(dst_h.at[g, wid], didx)
            gd = [None] * SEG_G
            sd = [None] * SEG_G
            for b in range(SEG_G):
                if b >= 2:
                    sd[b - 2].wait()
                gd[b] = pltpu.async_copy(table_h.at[sidx.at[b]],
                                         rows.at[b % 2], semg)
                if b >= 1:
                    gd[b - 1].wait()
                    sd[b - 1] = pltpu.async_copy(rows.at[(b - 1) % 2],
                                                 accum.at[didx.at[b - 1]],
                                                 sems, add=True)
            gd[SEG_G - 1].wait()
            sd[SEG_G - 1] = pltpu.async_copy(rows.at[(SEG_G - 1) % 2],
                                             accum.at[didx.at[SEG_G - 1]],
                                             sems, add=True)
            sd[SEG_G - 2].wait()
            sd[SEG_G - 1].wait()
            return carry

        lax.fori_loop(0, n_groups, group, 0)
        plsc.subcore_barrier()
        pltpu.sync_copy(accum.at[pl.ds(base, rows_per_tile)],
                        out_h.at[cid, pl.ds(base, rows_per_tile)])

    return pl.kernel(
        body,
        out_type=jax.ShapeDtypeStruct((NC, n_nodes_pad, 128), F32),
        mesh=_mesh(),
        scratch_types=[
            pltpu.VMEM((SEG_G, SUB), jnp.int32),
            pltpu.VMEM((SEG_G, SUB), jnp.int32),
            pltpu.VMEM((2, SUB, 128), F32),
            pltpu.VMEM_SHARED((n_nodes_pad, 128), F32),
            pltpu.SemaphoreType.DMA,
            pltpu.SemaphoreType.DMA,
        ],
    )(table, src4, dst4)


QC = 4


def _pool(table_w, idx4, n_rows_pad, specs, row_words, ch):
    i32 = jnp.int32
    P = len(specs)
    K = idx4.shape[0]
    n_quads = idx4.shape[1]
    assert n_rows_pad == n_quads * QC * ch
    per_worker = -(-n_quads // NW)
    m_hi = jnp.int32(-65536)

    def body(table_h, idx_h, out_h, ibuf, gbuf, zstage, semi, semg, semw):
        cid = lax.axis_index("c")
        sid = lax.axis_index("s")
        wid = sid * NC + cid

        def quad(t, carry):
            q = t * NW + wid

            @pl.when(q < n_quads)
            def _():
                descs = [pltpu.async_copy(idx_h.at[k, q], ibuf.at[k], semi)
                         for k in range(K)]
                for d in descs:
                    d.wait()

                gd = [None] * QC
                wd = [None] * QC

                def fire(c):
                    gd[c] = [pltpu.async_copy(
                        table_h.at[ibuf.at[k, c // 2,
                                           pl.ds(ch * (c % 2), ch)]],
                        gbuf.at[c % 2, k], semg) for k in range(K)]

                def drain_sum_wb(c):
                    slot = c % 2
                    for d in gd[c]:
                        d.wait()
                    if wd[c - 2] is not None:
                        for d in wd[c - 2]:
                            d.wait()

                    def srow(gg, carry2):
                        for p in range(P):
                            for j2 in range(8):
                                i = 2 * gg + (j2 // 4)
                                hi = None
                                lo = None
                                for (k, woff) in specs[p]:
                                    w = gbuf[slot, k, i,
                                             pl.ds(woff + 16 * (j2 % 4), 16)]
                                    h = jax.lax.bitcast_convert_type(
                                        w & m_hi, F32)
                                    l = jax.lax.bitcast_convert_type(
                                        w << 16, F32)
                                    hi = h if hi is None else hi + h
                                    lo = l if lo is None else lo + l
                                hb = jax.lax.bitcast_convert_type(hi, i32)
                                hb = hb + 0x7FFF + ((hb >> 16) & 1)
                                lb = jax.lax.bitcast_convert_type(lo, i32)
                                lb = lb + 0x7FFF + ((lb >> 16) & 1)
                                zstage[slot, p, gg, pl.ds(16 * j2, 16)] = (
                                    (hb & m_hi) | ((lb >> 16) & 0xFFFF))
                        return carry2

                    lax.fori_loop(0, ch // 2, srow, 0)
                    zrow0 = (q * QC + c) * (ch // 2)
                    wd[c] = [pltpu.async_copy(
                        zstage.at[slot, p],
                        out_h.at[p, pl.ds(zrow0, ch // 2)], semw)
                        for p in range(P)]

                fire(0)
                for c in range(1, QC):
                    fire(c)
                    drain_sum_wb(c - 1)
                drain_sum_wb(QC - 1)
                for d in wd[QC - 2]:
                    d.wait()
                for d in wd[QC - 1]:
                    d.wait()

            return carry

        lax.fori_loop(0, per_worker, quad, 0)

    return pl.kernel(
        body,
        out_type=jax.ShapeDtypeStruct((P, n_rows_pad // 2, 128), i32),
        mesh=_mesh(),
        scratch_types=[
            pltpu.VMEM((K, 2, 2 * ch), i32),
            pltpu.VMEM((2, K, ch, row_words), i32),
            pltpu.VMEM((2, P, ch // 2, 128), i32),
            pltpu.SemaphoreType.DMA,
            pltpu.SemaphoreType.DMA,
            pltpu.SemaphoreType.DMA,
        ],
    )(table_w, idx4)


def _tc_layer(xin, aggs, wself, wneigh, bias):
    n = xin.shape[0]
    blk = 1000

    def kfn(x_ref, a_ref, ws_ref, wn_ref, b_ref, o_ref):
        agg = a_ref[0] + a_ref[1]
        acc = jnp.dot(x_ref[...], ws_ref[...], preferred_element_type=F32)
        acc = acc + jnp.dot(agg, wn_ref[...], preferred_element_type=F32)
        o_ref[...] = jnp.maximum(acc + b_ref[...], 0.0)

    return pl.pallas_call(
        kfn,
        grid=(n // blk,),
        in_specs=[
            pl.BlockSpec((blk, 128), lambda i: (i, 0)),
            pl.BlockSpec((NC, blk, 128), lambda i: (0, i, 0)),
            pl.BlockSpec((128, 128), lambda i: (0, 0)),
            pl.BlockSpec((128, 128), lambda i: (0, 0)),
            pl.BlockSpec((1, 128), lambda i: (0, 0)),
        ],
        out_specs=pl.BlockSpec((blk, 128), lambda i: (i, 0)),
        out_shape=jax.ShapeDtypeStruct((n, 128), F32),
    )(xin, aggs, wself, wneigh, bias.reshape(1, 128))


def _tc_heads(h, aggs, ws2, wn2, b2, wa1, ba1, wa2, ba2, wb1, wg1, wp1, wi1):
    n = h.shape[0]
    blk = 1000

    def kfn(h_ref, a_ref, ws_ref, wn_ref, b_ref, wa1_ref, ba1_ref, wa2_ref,
            ba2_ref, wb1_ref, wg1_ref, wp1_ref, wi1_ref,
            at_ref, tb_ref, tg_ref, tp_ref, ti_ref):
        agg = a_ref[0] + a_ref[1]
        nr = jnp.dot(h_ref[...], ws_ref[...], preferred_element_type=F32)
        nr = nr + jnp.dot(agg, wn_ref[...], preferred_element_type=F32)
        nr = jnp.maximum(nr + b_ref[...], 0.0)
        t = jnp.maximum(jnp.dot(nr, wa1_ref[...], preferred_element_type=F32)
                        + ba1_ref[...], 0.0)
        at_ref[...] = jnp.dot(t, wa2_ref[...], preferred_element_type=F32) + ba2_ref[...]
        bf = jnp.bfloat16

        def proj(w_ref, j):
            return jnp.dot(nr, w_ref[pl.ds(128 * j, 128)],
                           preferred_element_type=F32).astype(bf)

        b0, b1 = proj(wb1_ref, 0), proj(wb1_ref, 1)
        tb_ref[:, 0:128] = b0
        tb_ref[:, 128:256] = b1
        g0, g1, g2 = (proj(wg1_ref, j) for j in range(3))
        tg_ref[0, :, 0:128] = g0
        tg_ref[0, :, 128:256] = g2
        tg_ref[1, :, 0:128] = g1
        tg_ref[1, :, 128:256] = g1
        p0, p1, p2, p3 = (proj(wp1_ref, j) for j in range(4))
        tp_ref[0, :, 0:128] = p0
        tp_ref[0, :, 128:256] = p3
        tp_ref[1, :, 0:128] = p1
        tp_ref[1, :, 128:256] = p2
        for j in range(4):
            ti_ref[:, pl.ds(128 * j, 128)] = proj(wi1_ref, j)

    full = lambda shape: pl.BlockSpec(shape, lambda i: tuple(0 for _ in shape))
    return pl.pallas_call(
        kfn,
        grid=(n // blk,),
        in_specs=[
            pl.BlockSpec((blk, 128), lambda i: (i, 0)),
            pl.BlockSpec((NC, blk, 128), lambda i: (0, i, 0)),
            full((128, 128)), full((128, 128)), full((1, 128)),
            full((128, 128)), full((1, 128)), full((128, 2)), full((1, 2)),
            full((256, 128)), full((384, 128)), full((512, 128)), full((512, 128)),
        ],
        out_specs=[
            pl.BlockSpec((blk, 2), lambda i: (i, 0)),
            pl.BlockSpec((blk, 256), lambda i: (i, 0)),
            pl.BlockSpec((2, blk, 256), lambda i: (0, i, 0)),
            pl.BlockSpec((2, blk, 256), lambda i: (0, i, 0)),
            pl.BlockSpec((blk, 512), lambda i: (i, 0)),
        ],
        out_shape=[
            jax.ShapeDtypeStruct((n, 2), F32),
            jax.ShapeDtypeStruct((n, 256), jnp.bfloat16),
            jax.ShapeDtypeStruct((2, n, 256), jnp.bfloat16),
            jax.ShapeDtypeStruct((2, n, 256), jnp.bfloat16),
            jax.ShapeDtypeStruct((n, 512), jnp.bfloat16),
        ],
    )(h, aggs, ws2, wn2, b2.reshape(1, 128), wa1, ba1.reshape(1, 128), wa2,
      ba2.reshape(1, 2), wb1, wg1, wp1, wi1)


def _tc_readout(z, w2, b1v, b2v, n_rows):
    P, rp, _ = z.shape
    out_d = w2.shape[1]
    blk = 512

    def kfn(z_ref, w2_ref, b1_ref, b2_ref, o_ref):
        acc = None
        for p in range(P):
            y = jnp.maximum(z_ref[p].astype(F32) + b1_ref[...], 0.0)
            yy = jnp.dot(y, w2_ref[...], preferred_element_type=F32)
            acc = yy if acc is None else acc + yy
        o_ref[...] = acc + float(P) * b2_ref[...]

    out = pl.pallas_call(
        kfn,
        grid=(rp // blk,),
        in_specs=[
            pl.BlockSpec((P, blk, 128), lambda i: (0, i, 0)),
            pl.BlockSpec((128, out_d), lambda i: (0, 0)),
            pl.BlockSpec((1, 128), lambda i: (0, 0)),
            pl.BlockSpec((1, out_d), lambda i: (0, 0)),
        ],
        out_specs=pl.BlockSpec((blk, out_d), lambda i: (i, 0)),
        out_shape=jax.ShapeDtypeStruct((rp, out_d), F32),
    )(z, w2, b1v.reshape(1, 128), b2v.reshape(1, out_d))
    return out[:n_rows]


def _words(t, w):
    return jax.lax.bitcast_convert_type(
        t.reshape(t.shape[0], w, 2), jnp.int32)


def _readout(table_w, idx_cols, specs, w2, b1v, b2v, ch):
    r = idx_cols[0].shape[0]
    row_words = table_w.shape[1]
    rp = _ceil_to(r, max(QC * ch * 2, 512))
    idx = jnp.stack(idx_cols)
    idx = jnp.pad(idx, ((0, 0), (0, rp - r)))
    idx4 = idx.reshape(len(idx_cols), rp // (QC * ch), 2, 2 * ch)
    zw = _pool(table_w, idx4, rp, specs, row_words, ch)
    z = jax.lax.bitcast_convert_type(zw, jnp.bfloat16).reshape(
        len(specs), rp, 128)
    return _tc_readout(z, w2, b1v, b2v, r)


def kernel(x, edge_index, bonds, angles, propers, impropers, Ws1, Wn1, b1,
           Ws2, Wn2, b2, Wa1, ba1, Wa2, ba2, Wb1, bb1, Wb2, bb2, Wg1, bg1,
           Wg2, bg2, Wp1, bp1, Wp2, bp2, Wi1, bi1, Wi2, bi2):
    n = x.shape[0]
    e = edge_index.shape[1]
    n_pad = _ceil_to(n + 1, NS * 8)
    e_pad = _ceil_to(e, SEG_G * SUB * NW)

    src = jnp.pad(edge_index[0], (0, e_pad - e))
    dst = jnp.pad(edge_index[1], (0, e_pad - e), constant_values=n)
    n_groups = e_pad // (SEG_G * SUB * NW)
    src4 = src.reshape(n_groups, NW, SEG_G, SUB)
    dst4 = dst.reshape(n_groups, NW, SEG_G, SUB)

    agg1 = _segsum(x, src4, dst4, n_pad)[:, :n]
    h = _tc_layer(x, agg1, Ws1, Wn1, b1)
    agg2 = _segsum(h, src4, dst4, n_pad)[:, :n]
    atoms, tb, tg, tp, ti = _tc_heads(h, agg2, Ws2, Wn2, b2, Wa1, ba1, Wa2,
                                      ba2, Wb1, Wg1, Wp1, Wi1)

    tbw = _words(tb, 128)
    tgw = _words(tg.reshape(2 * n, 256), 128)
    tpw = _words(tp.reshape(2 * n, 256), 128)
    tiw = _words(ti, 256)

    b0c, b1c = bonds[:, 0], bonds[:, 1]
    bonds_out = _readout(
        tbw, [b0c, b1c],
        [[(0, 0), (1, 64)], [(1, 0), (0, 64)]],
        Wb2, bb1, bb2, 64)
    a0c, a1c, a2c = angles[:, 0], angles[:, 1], angles[:, 2]
    angles_out = _readout(
        tgw, [a0c, n + a1c, a2c],
        [[(0, 0), (1, 0), (2, 64)], [(2, 0), (1, 0), (0, 64)]],
        Wg2, bg1, bg2, 64)
    p0c, p1c, p2c, p3c = (propers[:, j] for j in range(4))
    propers_out = _readout(
        tpw, [p0c, n + p1c, n + p2c, p3c],
        [[(0, 0), (1, 0), (2, 64), (3, 64)],
         [(3, 0), (2, 0), (1, 64), (0, 64)]],
        Wp2, bp1, bp2, 64)
    i0c, i1c, i2c, i3c = (impropers[:, j] for j in range(4))
    impropers_out = _readout(
        tiw, [i0c, i1c, i2c, i3c],
        [[(0, 0), (1, 64), (2, 128), (3, 192)],
         [(2, 0), (1, 64), (3, 128), (0, 192)],
         [(3, 0), (1, 64), (0, 128), (2, 192)]],
        Wi2, bi1, bi2, 32)
    return (atoms, bonds_out, angles_out, propers_out, impropers_out)


# speedup vs baseline: 13.2893x; 13.2893x over previous
import functools

import jax
import jax.numpy as jnp
from jax import lax
from jax.experimental import pallas as pl
from jax.experimental.pallas import tpu as pltpu
from jax.experimental.pallas import tpu_sc as plsc

NC = 2
NS = 16
NW = NC * NS
SUB = 128
CH = 2 * SUB
F32 = jnp.float32


def _ceil_to(v, m):
    return -(-v // m) * m


def _mesh():
    return plsc.VectorSubcoreMesh(core_axis_name="c", subcore_axis_name="s")


SEG_G = 20


def _segsum(table, src4, dst4, n_nodes_pad):
    n_groups = src4.shape[0]
    rows_per_tile = n_nodes_pad // NS

    def body(table_h, src_h, dst_h, out_h, sidx, didx, rows, accum, semg, sems):
        cid = lax.axis_index("c")
        sid = lax.axis_index("s")
        wid = sid * NC + cid

        def zrow(i, carry):
            for j in range(8):
                rows[0, i, pl.ds(16 * j, 16)] = jnp.zeros((16,), F32)
            return carry

        lax.fori_loop(0, SUB, zrow, 0)
        base = sid * rows_per_tile
        off = 0
        while off < rows_per_tile:
            n = min(SUB, rows_per_tile - off)
            pltpu.sync_copy(rows.at[0, pl.ds(0, n)], accum.at[pl.ds(base + off, n)])
            off += n
        plsc.subcore_barrier()

        def group(g, carry):
            pltpu.sync_copy(src_h.at[g, wid], sidx)
            pltpu.sync_copy(dst_h.at[g, wid], didx)
            gd = [None] * SEG_G
            sd = [None] * SEG_G
            for b in range(SEG_G):
                if b >= 2:
                    sd[b - 2].wait()
                gd[b] = pltpu.async_copy(table_h.at[sidx.at[b]],
                                         rows.at[b % 2], semg)
                if b >= 1:
                    gd[b - 1].wait()
                    sd[b - 1] = pltpu.async_copy(rows.at[(b - 1) % 2],
                                                 accum.at[didx.at[b - 1]],
                                                 sems, add=True)
            gd[SEG_G - 1].wait()
            sd[SEG_G - 1] = pltpu.async_copy(rows.at[(SEG_G - 1) % 2],
                                             accum.at[didx.at[SEG_G - 1]],
                                             sems, add=True)
            sd[SEG_G - 2].wait()
            sd[SEG_G - 1].wait()
            return carry

        lax.fori_loop(0, n_groups, group, 0)
        plsc.subcore_barrier()
        pltpu.sync_copy(accum.at[pl.ds(base, rows_per_tile)],
                        out_h.at[cid, pl.ds(base, rows_per_tile)])

    return pl.kernel(
        body,
        out_type=jax.ShapeDtypeStruct((NC, n_nodes_pad, 128), F32),
        mesh=_mesh(),
        scratch_types=[
            pltpu.VMEM((SEG_G, SUB), jnp.int32),
            pltpu.VMEM((SEG_G, SUB), jnp.int32),
            pltpu.VMEM((2, SUB, 128), F32),
            pltpu.VMEM_SHARED((n_nodes_pad, 128), F32),
            pltpu.SemaphoreType.DMA,
            pltpu.SemaphoreType.DMA,
        ],
    )(table, src4, dst4)


QC = 4


def _pool(table_w, idx4, n_rows_pad, specs, row_words, ch):
    i32 = jnp.int32
    P = len(specs)
    K = idx4.shape[0]
    n_quads = idx4.shape[1]
    assert n_rows_pad == n_quads * QC * ch
    per_worker = -(-n_quads // NW)

    def body(table_h, idx_h, out_h, ibuf, gbuf, zstage, semi, semg, semw):
        cid = lax.axis_index("c")
        sid = lax.axis_index("s")
        wid = sid * NC + cid

        def quad(t, carry):
            q = t * NW + wid

            @pl.when(q < n_quads)
            def _():
                descs = [pltpu.async_copy(idx_h.at[k, q], ibuf.at[k], semi)
                         for k in range(K)]
                for d in descs:
                    d.wait()

                gd = [None] * QC
                wd = [None] * QC

                def fire(c):
                    gd[c] = [pltpu.async_copy(
                        table_h.at[ibuf.at[k, c // 2,
                                           pl.ds(ch * (c % 2), ch)]],
                        gbuf.at[c % 2, k], semg) for k in range(K)]

                def drain_sum_wb(c):
                    slot = c % 2
                    for d in gd[c]:
                        d.wait()
                    if wd[c - 2] is not None:
                        for d in wd[c - 2]:
                            d.wait()

                    def srow(gg, carry2):
                        for p in range(P):
                            for j2 in range(8):
                                i = 2 * gg + (j2 // 4)
                                hi = None
                                lo = None
                                for (k, woff) in specs[p]:
                                    w = gbuf[slot, k, i,
                                             pl.ds(woff + 16 * (j2 % 4), 16)]
                                    h = jax.lax.bitcast_convert_type(
                                        w & (-65536), F32)
                                    l = jax.lax.bitcast_convert_type(
                                        w << 16, F32)
                                    hi = h if hi is None else hi + h
                                    lo = l if lo is None else lo + l
                                hb = jax.lax.bitcast_convert_type(hi, i32)
                                hb = hb + 0x7FFF + ((hb >> 16) & 1)
                                lb = jax.lax.bitcast_convert_type(lo, i32)
                                lb = lb + 0x7FFF + ((lb >> 16) & 1)
                                zstage[slot, p, gg, pl.ds(16 * j2, 16)] = (
                                    (hb & (-65536)) | ((lb >> 16) & 0xFFFF))
                        return carry2

                    lax.fori_loop(0, ch // 2, srow, 0)
                    zrow0 = (q * QC + c) * (ch // 2)
                    wd[c] = [pltpu.async_copy(
                        zstage.at[slot, p],
                        out_h.at[p, pl.ds(zrow0, ch // 2)], semw)
                        for p in range(P)]

                fire(0)
                for c in range(1, QC):
                    fire(c)
                    drain_sum_wb(c - 1)
                drain_sum_wb(QC - 1)
                for d in wd[QC - 2]:
                    d.wait()
                for d in wd[QC - 1]:
                    d.wait()

            return carry

        lax.fori_loop(0, per_worker, quad, 0)

    return pl.kernel(
        body,
        out_type=jax.ShapeDtypeStruct((P, n_rows_pad // 2, 128), i32),
        mesh=_mesh(),
        scratch_types=[
            pltpu.VMEM((K, 2, 2 * ch), i32),
            pltpu.VMEM((2, K, ch, row_words), i32),
            pltpu.VMEM((2, P, ch // 2, 128), i32),
            pltpu.SemaphoreType.DMA,
            pltpu.SemaphoreType.DMA,
            pltpu.SemaphoreType.DMA,
        ],
    )(table_w, idx4)


def _tc_layer(xin, aggs, wself, wneigh, bias):
    n = xin.shape[0]
    blk = 1000

    def kfn(x_ref, a_ref, ws_ref, wn_ref, b_ref, o_ref):
        agg = a_ref[0] + a_ref[1]
        acc = jnp.dot(x_ref[...], ws_ref[...], preferred_element_type=F32)
        acc = acc + jnp.dot(agg, wn_ref[...], preferred_element_type=F32)
        o_ref[...] = jnp.maximum(acc + b_ref[...], 0.0)

    return pl.pallas_call(
        kfn,
        grid=(n // blk,),
        in_specs=[
            pl.BlockSpec((blk, 128), lambda i: (i, 0)),
            pl.BlockSpec((NC, blk, 128), lambda i: (0, i, 0)),
            pl.BlockSpec((128, 128), lambda i: (0, 0)),
            pl.BlockSpec((128, 128), lambda i: (0, 0)),
            pl.BlockSpec((1, 128), lambda i: (0, 0)),
        ],
        out_specs=pl.BlockSpec((blk, 128), lambda i: (i, 0)),
        out_shape=jax.ShapeDtypeStruct((n, 128), F32),
    )(xin, aggs, wself, wneigh, bias.reshape(1, 128))


def _tc_heads(h, aggs, ws2, wn2, b2, wa1, ba1, wa2, ba2, wb1, wg1, wp1, wi1):
    n = h.shape[0]
    blk = 1000

    def kfn(h_ref, a_ref, ws_ref, wn_ref, b_ref, wa1_ref, ba1_ref, wa2_ref,
            ba2_ref, wb1_ref, wg1_ref, wp1_ref, wi1_ref,
            at_ref, tb_ref, tg_ref, tp_ref, ti_ref):
        agg = a_ref[0] + a_ref[1]
        nr = jnp.dot(h_ref[...], ws_ref[...], preferred_element_type=F32)
        nr = nr + jnp.dot(agg, wn_ref[...], preferred_element_type=F32)
        nr = jnp.maximum(nr + b_ref[...], 0.0)
        t = jnp.maximum(jnp.dot(nr, wa1_ref[...], preferred_element_type=F32)
                        + ba1_ref[...], 0.0)
        at_ref[...] = jnp.dot(t, wa2_ref[...], preferred_element_type=F32) + ba2_ref[...]
        def proj(w_ref, j):
            return jnp.dot(nr, w_ref[pl.ds(128 * j, 128)],
                           preferred_element_type=F32)

        def pack2(v):
            bits = jax.lax.bitcast_convert_type(v, jnp.int32)
            a = bits[:, 0:64]
            b = bits[:, 64:128]
            a = a + 0x7FFF + ((a >> 16) & 1)
            b = b + 0x7FFF + ((b >> 16) & 1)
            return ((a >> 16) & 0xFFFF) | (b & (-65536))

        b0, b1 = proj(wb1_ref, 0), proj(wb1_ref, 1)
        tb_ref[:, 0:64] = pack2(b0)
        tb_ref[:, 64:128] = pack2(b1)
        g0, g1, g2 = (proj(wg1_ref, j) for j in range(3))
        tg_ref[0, :, 0:64] = pack2(g0)
        tg_ref[0, :, 64:128] = pack2(g2)
        g1p = pack2(g1)
        tg_ref[1, :, 0:64] = g1p
        tg_ref[1, :, 64:128] = g1p
        p0, p1, p2, p3 = (proj(wp1_ref, j) for j in range(4))
        tp_ref[0, :, 0:64] = pack2(p0)
        tp_ref[0, :, 64:128] = pack2(p3)
        tp_ref[1, :, 0:64] = pack2(p1)
        tp_ref[1, :, 64:128] = pack2(p2)
        for j in range(4):
            ti_ref[:, pl.ds(64 * j, 64)] = pack2(proj(wi1_ref, j))

    full = lambda shape: pl.BlockSpec(shape, lambda i: tuple(0 for _ in shape))
    return pl.pallas_call(
        kfn,
        grid=(n // blk,),
        in_specs=[
            pl.BlockSpec((blk, 128), lambda i: (i, 0)),
            pl.BlockSpec((NC, blk, 128), lambda i: (0, i, 0)),
            full((128, 128)), full((128, 128)), full((1, 128)),
            full((128, 128)), full((1, 128)), full((128, 2)), full((1, 2)),
            full((256, 128)), full((384, 128)), full((512, 128)), full((512, 128)),
        ],
        out_specs=[
            pl.BlockSpec((blk, 2), lambda i: (i, 0)),
            pl.BlockSpec((blk, 128), lambda i: (i, 0)),
            pl.BlockSpec((2, blk, 128), lambda i: (0, i, 0)),
            pl.BlockSpec((2, blk, 128), lambda i: (0, i, 0)),
            pl.BlockSpec((blk, 256), lambda i: (i, 0)),
        ],
        out_shape=[
            jax.ShapeDtypeStruct((n, 2), F32),
            jax.ShapeDtypeStruct((n, 128), jnp.int32),
            jax.ShapeDtypeStruct((2, n, 128), jnp.int32),
            jax.ShapeDtypeStruct((2, n, 128), jnp.int32),
            jax.ShapeDtypeStruct((n, 256), jnp.int32),
        ],
    )(h, aggs, ws2, wn2, b2.reshape(1, 128), wa1, ba1.reshape(1, 128), wa2,
      ba2.reshape(1, 2), wb1, wg1, wp1, wi1)


def _tc_readout(zw, w2, b1v, b2v, n_rows):
    P, rp2, _ = zw.shape
    out_d = w2.shape[1]
    blk = 256

    def kfn(z_ref, w2_ref, b1_ref, b2_ref, o_ref):
        for g in range(2):
            acc = None
            for p in range(P):
                ww = z_ref[p, :, pl.ds(64 * g, 64)]
                lo = jax.lax.bitcast_convert_type(ww << 16, F32)
                hi = jax.lax.bitcast_convert_type(ww & (-65536), F32)
                y = jnp.maximum(lo + b1_ref[:, 0:64], 0.0)
                yy = jnp.dot(y, w2_ref[0:64], preferred_element_type=F32)
                y = jnp.maximum(hi + b1_ref[:, 64:128], 0.0)
                yy = yy + jnp.dot(y, w2_ref[64:128],
                                  preferred_element_type=F32)
                acc = yy if acc is None else acc + yy
            o_ref[:, g, :] = acc + float(P) * b2_ref[...]

    out = pl.pallas_call(
        kfn,
        grid=(rp2 // blk,),
        in_specs=[
            pl.BlockSpec((P, blk, 128), lambda i: (0, i, 0)),
            pl.BlockSpec((128, out_d), lambda i: (0, 0)),
            pl.BlockSpec((1, 128), lambda i: (0, 0)),
            pl.BlockSpec((1, out_d), lambda i: (0, 0)),
        ],
        out_specs=pl.BlockSpec((blk, 2, out_d), lambda i: (i, 0, 0)),
        out_shape=jax.ShapeDtypeStruct((rp2, 2, out_d), F32),
    )(zw, w2, b1v.reshape(1, 128), b2v.reshape(1, out_d))
    return out.reshape(2 * rp2, out_d)[:n_rows]


def _readout(table_w, idx_cols, specs, w2, b1v, b2v, ch):
    r = idx_cols[0].shape[0]
    row_words = table_w.shape[1]
    rp = _ceil_to(r, max(QC * ch * 2, 512))
    idx = jnp.stack(idx_cols)
    idx = jnp.pad(idx, ((0, 0), (0, rp - r)))
    idx4 = idx.reshape(len(idx_cols), rp // (QC * ch), 2, 2 * ch)
    zw = _pool(table_w, idx4, rp, specs, row_words, ch)
    return _tc_readout(zw, w2, b1v, b2v, r)


def kernel(x, edge_index, bonds, angles, propers, impropers, Ws1, Wn1, b1,
           Ws2, Wn2, b2, Wa1, ba1, Wa2, ba2, Wb1, bb1, Wb2, bb2, Wg1, bg1,
           Wg2, bg2, Wp1, bp1, Wp2, bp2, Wi1, bi1, Wi2, bi2):
    n = x.shape[0]
    e = edge_index.shape[1]
    n_pad = _ceil_to(n + 1, NS * 8)
    e_pad = _ceil_to(e, SEG_G * SUB * NW)

    src = jnp.pad(edge_index[0], (0, e_pad - e))
    dst = jnp.pad(edge_index[1], (0, e_pad - e), constant_values=n)
    n_groups = e_pad // (SEG_G * SUB * NW)
    src4 = src.reshape(n_groups, NW, SEG_G, SUB)
    dst4 = dst.reshape(n_groups, NW, SEG_G, SUB)

    agg1 = _segsum(x, src4, dst4, n_pad)[:, :n]
    h = _tc_layer(x, agg1, Ws1, Wn1, b1)
    agg2 = _segsum(h, src4, dst4, n_pad)[:, :n]
    atoms, tb, tg, tp, ti = _tc_heads(h, agg2, Ws2, Wn2, b2, Wa1, ba1, Wa2,
                                      ba2, Wb1, Wg1, Wp1, Wi1)

    tbw = tb
    tgw = tg.reshape(2 * n, 128)
    tpw = tp.reshape(2 * n, 128)
    tiw = ti

    b0c, b1c = bonds[:, 0], bonds[:, 1]
    bonds_out = _readout(
        tbw, [b0c, b1c],
        [[(0, 0), (1, 64)], [(1, 0), (0, 64)]],
        Wb2, bb1, bb2, 64)
    a0c, a1c, a2c = angles[:, 0], angles[:, 1], angles[:, 2]
    angles_out = _readout(
        tgw, [a0c, n + a1c, a2c],
        [[(0, 0), (1, 0), (2, 64)], [(2, 0), (1, 0), (0, 64)]],
        Wg2, bg1, bg2, 64)
    p0c, p1c, p2c, p3c = (propers[:, j] for j in range(4))
    propers_out = _readout(
        tpw, [p0c, n + p1c, n + p2c, p3c],
        [[(0, 0), (1, 0), (2, 64), (3, 64)],
         [(3, 0), (2, 0), (1, 64), (0, 64)]],
        Wp2, bp1, bp2, 64)
    i0c, i1c, i2c, i3c = (impropers[:, j] for j in range(4))
    impropers_out = _readout(
        tiw, [i0c, i1c, i2c, i3c],
        [[(0, 0), (1, 64), (2, 128), (3, 192)],
         [(2, 0), (1, 64), (3, 128), (0, 192)],
         [(3, 0), (1, 64), (0, 128), (2, 192)]],
        Wi2, bi1, bi2, 32)
    return (atoms, bonds_out, angles_out, propers_out, impropers_out)


# speedup vs baseline: 14.0359x; 1.0562x over previous
import functools

import jax
import jax.numpy as jnp
from jax import lax
from jax.experimental import pallas as pl
from jax.experimental.pallas import tpu as pltpu
from jax.experimental.pallas import tpu_sc as plsc

NC = 2
NS = 16
NW = NC * NS
SUB = 128
CH = 2 * SUB
F32 = jnp.float32


def _ceil_to(v, m):
    return -(-v // m) * m


def _mesh():
    return plsc.VectorSubcoreMesh(core_axis_name="c", subcore_axis_name="s")


SEG_G = 20


def _segsum(table, src4, dst4, n_nodes_pad):
    n_groups = src4.shape[0]
    rows_per_tile = n_nodes_pad // NS

    def body(table_h, src_h, dst_h, out_h, sidx, didx, rows, accum, semg, sems):
        cid = lax.axis_index("c")
        sid = lax.axis_index("s")
        wid = sid * NC + cid

        def zrow(i, carry):
            for j in range(8):
                rows[0, i, pl.ds(16 * j, 16)] = jnp.zeros((16,), F32)
            return carry

        lax.fori_loop(0, SUB, zrow, 0)
        base = sid * rows_per_tile
        off = 0
        while off < rows_per_tile:
            n = min(SUB, rows_per_tile - off)
            pltpu.sync_copy(rows.at[0, pl.ds(0, n)], accum.at[pl.ds(base + off, n)])
            off += n
        plsc.subcore_barrier()

        def group(g, carry):
            pltpu.sync_copy(src_h.at[g, wid], sidx)
            pltpu.sync_copy(dst_h.at[g, wid], didx)
            gd = [None] * SEG_G
            sd = [None] * SEG_G
            for b in range(SEG_G):
                if b >= 2:
                    sd[b - 2].wait()
                gd[b] = pltpu.async_copy(table_h.at[sidx.at[b]],
                                         rows.at[b % 2], semg)
                if b >= 1:
                    gd[b - 1].wait()
                    sd[b - 1] = pltpu.async_copy(rows.at[(b - 1) % 2],
                                                 accum.at[didx.at[b - 1]],
                                                 sems, add=True)
            gd[SEG_G - 1].wait()
            sd[SEG_G - 1] = pltpu.async_copy(rows.at[(SEG_G - 1) % 2],
                                             accum.at[didx.at[SEG_G - 1]],
                                             sems, add=True)
            sd[SEG_G - 2].wait()
            sd[SEG_G - 1].wait()
            return carry

        lax.fori_loop(0, n_groups, group, 0)
        plsc.subcore_barrier()
        pltpu.sync_copy(accum.at[pl.ds(base, rows_per_tile)],
                        out_h.at[cid, pl.ds(base, rows_per_tile)])

    return pl.kernel(
        body,
        out_type=jax.ShapeDtypeStruct((NC, n_nodes_pad, 128), F32),
        mesh=_mesh(),
        scratch_types=[
            pltpu.VMEM((SEG_G, SUB), jnp.int32),
            pltpu.VMEM((SEG_G, SUB), jnp.int32),
            pltpu.VMEM((2, SUB, 128), F32),
            pltpu.VMEM_SHARED((n_nodes_pad, 128), F32),
            pltpu.SemaphoreType.DMA,
            pltpu.SemaphoreType.DMA,
        ],
    )(table, src4, dst4)


def _pool(table, idx4, n_rows_pad, n_perms, k_slots):
    n_chunks = n_rows_pad // CH
    per_worker = -(-n_chunks // NW)
    P, K = n_perms, k_slots

    def body(table_h, idx_h, out_h, ibuf, zbuf, sem):
        cid = lax.axis_index("c")
        sid = lax.axis_index("s")
        wid = sid * NC + cid

        def chunk(t, carry):
            ci = t * NW + wid

            @pl.when(ci < n_chunks)
            def _():
                descs = []
                for p in range(P):
                    for k in range(K):
                        descs.append(pltpu.async_copy(idx_h.at[p, k, ci],
                                                      ibuf.at[p, k], sem))
                for d in descs:
                    d.wait()
                for k in range(K):
                    descs = []
                    for p in range(P):
                        for j in range(2):
                            descs.append(pltpu.async_copy(
                                table_h.at[ibuf.at[p, k, j]],
                                zbuf.at[p, pl.ds(SUB * j, SUB)],
                                sem, add=(k > 0)))
                    for d in descs:
                        d.wait()
                descs = []
                for p in range(P):
                    descs.append(pltpu.async_copy(
                        zbuf.at[p], out_h.at[p, pl.ds(ci * CH, CH)], sem))
                for d in descs:
                    d.wait()

            return carry

        lax.fori_loop(0, per_worker, chunk, 0)

    return pl.kernel(
        body,
        out_type=jax.ShapeDtypeStruct((P, n_rows_pad, 128), jnp.float32),
        mesh=_mesh(),
        scratch_types=[
            pltpu.VMEM((P, K, 2, SUB), jnp.int32),
            pltpu.VMEM((P, CH, 128), jnp.float32),
            pltpu.SemaphoreType.DMA,
        ],
    )(table, idx4)


def _tc_layer(xin, aggs, wself, wneigh, bias):
    n = xin.shape[0]
    blk = 1000

    def kfn(x_ref, a_ref, ws_ref, wn_ref, b_ref, o_ref):
        agg = a_ref[0] + a_ref[1]
        acc = jnp.dot(x_ref[...], ws_ref[...], preferred_element_type=F32)
        acc = acc + jnp.dot(agg, wn_ref[...], preferred_element_type=F32)
        o_ref[...] = jnp.maximum(acc + b_ref[...], 0.0)

    return pl.pallas_call(
        kfn,
        grid=(n // blk,),
        in_specs=[
            pl.BlockSpec((blk, 128), lambda i: (i, 0)),
            pl.BlockSpec((NC, blk, 128), lambda i: (0, i, 0)),
            pl.BlockSpec((128, 128), lambda i: (0, 0)),
            pl.BlockSpec((128, 128), lambda i: (0, 0)),
            pl.BlockSpec((1, 128), lambda i: (0, 0)),
        ],
        out_specs=pl.BlockSpec((blk, 128), lambda i: (i, 0)),
        out_shape=jax.ShapeDtypeStruct((n, 128), F32),
    )(xin, aggs, wself, wneigh, bias.reshape(1, 128))


def _tc_heads(h, aggs, ws2, wn2, b2, wa1, ba1, wa2, ba2, wb1, wg1, wp1, wi1):
    n = h.shape[0]
    blk = 1000

    def kfn(h_ref, a_ref, ws_ref, wn_ref, b_ref, wa1_ref, ba1_ref, wa2_ref,
            ba2_ref, wb1_ref, wg1_ref, wp1_ref, wi1_ref,
            at_ref, tb_ref, tg_ref, tp_ref, ti_ref):
        agg = a_ref[0] + a_ref[1]
        nr = jnp.dot(h_ref[...], ws_ref[...], preferred_element_type=F32)
        nr = nr + jnp.dot(agg, wn_ref[...], preferred_element_type=F32)
        nr = jnp.maximum(nr + b_ref[...], 0.0)
        t = jnp.maximum(jnp.dot(nr, wa1_ref[...], preferred_element_type=F32)
                        + ba1_ref[...], 0.0)
        at_ref[...] = jnp.dot(t, wa2_ref[...], preferred_element_type=F32) + ba2_ref[...]
        for j in range(2):
            tb_ref[j] = jnp.dot(nr, wb1_ref[pl.ds(128 * j, 128)],
                                preferred_element_type=F32)
        for j in range(3):
            tg_ref[j] = jnp.dot(nr, wg1_ref[pl.ds(128 * j, 128)],
                                preferred_element_type=F32)
        for j in range(4):
            tp_ref[j] = jnp.dot(nr, wp1_ref[pl.ds(128 * j, 128)],
                                preferred_element_type=F32)
        for j in range(4):
            ti_ref[j] = jnp.dot(nr, wi1_ref[pl.ds(128 * j, 128)],
                                preferred_element_type=F32)

    full = lambda shape: pl.BlockSpec(shape, lambda i: tuple(0 for _ in shape))
    return pl.pallas_call(
        kfn,
        grid=(n // blk,),
        in_specs=[
            pl.BlockSpec((blk, 128), lambda i: (i, 0)),
            pl.BlockSpec((NC, blk, 128), lambda i: (0, i, 0)),
            full((128, 128)), full((128, 128)), full((1, 128)),
            full((128, 128)), full((1, 128)), full((128, 2)), full((1, 2)),
            full((256, 128)), full((384, 128)), full((512, 128)), full((512, 128)),
        ],
        out_specs=[
            pl.BlockSpec((blk, 2), lambda i: (i, 0)),
            pl.BlockSpec((2, blk, 128), lambda i: (0, i, 0)),
            pl.BlockSpec((3, blk, 128), lambda i: (0, i, 0)),
            pl.BlockSpec((4, blk, 128), lambda i: (0, i, 0)),
            pl.BlockSpec((4, blk, 128), lambda i: (0, i, 0)),
        ],
        out_shape=[
            jax.ShapeDtypeStruct((n, 2), F32),
            jax.ShapeDtypeStruct((2, n, 128), F32),
            jax.ShapeDtypeStruct((3, n, 128), F32),
            jax.ShapeDtypeStruct((4, n, 128), F32),
            jax.ShapeDtypeStruct((4, n, 128), F32),
        ],
    )(h, aggs, ws2, wn2, b2.reshape(1, 128), wa1, ba1.reshape(1, 128), wa2,
      ba2.reshape(1, 2), wb1, wg1, wp1, wi1)


def _tc_readout(z, w2, b1v, b2v, n_rows):
    P, rp, _ = z.shape
    out_d = w2.shape[1]
    blk = 512

    def kfn(z_ref, w2_ref, b1_ref, b2_ref, o_ref):
        acc = None
        for p in range(P):
            y = jnp.maximum(z_ref[p] + b1_ref[...], 0.0)
            yy = jnp.dot(y, w2_ref[...], preferred_element_type=F32)
            acc = yy if acc is None else acc + yy
        o_ref[...] = acc + float(P) * b2_ref[...]

    out = pl.pallas_call(
        kfn,
        grid=(rp // blk,),
        in_specs=[
            pl.BlockSpec((P, blk, 128), lambda i: (0, i, 0)),
            pl.BlockSpec((128, out_d), lambda i: (0, 0)),
            pl.BlockSpec((1, 128), lambda i: (0, 0)),
            pl.BlockSpec((1, out_d), lambda i: (0, 0)),
        ],
        out_specs=pl.BlockSpec((blk, out_d), lambda i: (i, 0)),
        out_shape=jax.ShapeDtypeStruct((rp, out_d), F32),
    )(z, w2, b1v.reshape(1, 128), b2v.reshape(1, out_d))
    return out[:n_rows]


def _readout(table, inter, perms, n_nodes, w2, b1v, b2v):
    r, _ = inter.shape
    P = len(perms)
    K = len(perms[0])
    rp = _ceil_to(r, 512)
    cols = []
    for perm in perms:
        cols.append(jnp.stack([inter[:, perm[j]] + j * n_nodes for j in range(K)]))
    idx = jnp.stack(cols)
    idx = jnp.pad(idx, ((0, 0), (0, 0), (0, rp - r)))
    idx4 = idx.reshape(P, K, rp // CH, 2, SUB)
    z = _pool(table, idx4, rp, P, K)
    return _tc_readout(z, w2, b1v, b2v, r)


def kernel(x, edge_index, bonds, angles, propers, impropers, Ws1, Wn1, b1,
           Ws2, Wn2, b2, Wa1, ba1, Wa2, ba2, Wb1, bb1, Wb2, bb2, Wg1, bg1,
           Wg2, bg2, Wp1, bp1, Wp2, bp2, Wi1, bi1, Wi2, bi2):
    n = x.shape[0]
    e = edge_index.shape[1]
    n_pad = _ceil_to(n + 1, NS * 8)
    e_pad = _ceil_to(e, SEG_G * SUB * NW)

    src = jnp.pad(edge_index[0], (0, e_pad - e))
    dst = jnp.pad(edge_index[1], (0, e_pad - e), constant_values=n)
    n_groups = e_pad // (SEG_G * SUB * NW)
    src4 = src.reshape(n_groups, NW, SEG_G, SUB)
    dst4 = dst.reshape(n_groups, NW, SEG_G, SUB)

    agg1 = _segsum(x, src4, dst4, n_pad)[:, :n]
    h = _tc_layer(x, agg1, Ws1, Wn1, b1)
    agg2 = _segsum(h, src4, dst4, n_pad)[:, :n]
    atoms, tb, tg, tp, ti = _tc_heads(h, agg2, Ws2, Wn2, b2, Wa1, ba1, Wa2,
                                      ba2, Wb1, Wg1, Wp1, Wi1)

    bonds_out = _readout(tb.reshape(2 * n, 128), bonds, [(0, 1), (1, 0)],
                         n, Wb2, bb1, bb2)
    angles_out = _readout(tg.reshape(3 * n, 128), angles,
                          [(0, 1, 2), (2, 1, 0)], n, Wg2, bg1, bg2)
    propers_out = _readout(tp.reshape(4 * n, 128), propers,
                           [(0, 1, 2, 3), (3, 2, 1, 0)], n, Wp2, bp1, bp2)
    imp_perms = [(0, 1, 2, 3), (2, 1, 3, 0), (3, 1, 0, 2)]
    impropers_out = _readout(ti.reshape(4 * n, 128), impropers, imp_perms,
                             n, Wi2, bi1, bi2)
    return (atoms, bonds_out, angles_out, propers_out, impropers_out)


# speedup vs baseline: 14.0466x; 1.0008x over previous
import jax
import jax.numpy as jnp
from jax import lax
from jax.experimental import pallas as pl
from jax.experimental.pallas import tpu as pltpu
from jax.experimental.pallas import tpu_sc as plsc

NC = 2
NS = 16
NW = NC * NS
SUB = 128
CH = 2 * SUB
F32 = jnp.float32


def _ceil_to(v, m):
    return -(-v // m) * m


def _mesh():
    return plsc.VectorSubcoreMesh(core_axis_name="c", subcore_axis_name="s")


SEG_G = 20


def _segsum(table, src4, dst4, n_nodes_pad):
    n_groups = src4.shape[0]
    rows_per_tile = n_nodes_pad // NS

    def body(table_h, src_h, dst_h, out_h, sidx, didx, rows, accum, semg, sems):
        cid = lax.axis_index("c")
        sid = lax.axis_index("s")
        wid = sid * NC + cid

        def zrow(i, carry):
            for j in range(8):
                rows[0, i, pl.ds(16 * j, 16)] = jnp.zeros((16,), F32)
            return carry

        lax.fori_loop(0, SUB, zrow, 0)
        base = sid * rows_per_tile
        off = 0
        while off < rows_per_tile:
            n = min(SUB, rows_per_tile - off)
            pltpu.sync_copy(rows.at[0, pl.ds(0, n)], accum.at[pl.ds(base + off, n)])
            off += n
        plsc.subcore_barrier()

        def group(g, carry):
            pltpu.sync_copy(src_h.at[g, wid], sidx)
            pltpu.sync_copy(dst_h.at[g, wid], didx)
            gd = [None] * SEG_G
            sd = [None] * SEG_G
            for b in range(SEG_G):
                if b >= 2:
                    sd[b - 2].wait()
                gd[b] = pltpu.async_copy(table_h.at[sidx.at[b]],
                                         rows.at[b % 2], semg)
                if b >= 1:
                    gd[b - 1].wait()
                    sd[b - 1] = pltpu.async_copy(rows.at[(b - 1) % 2],
                                                 accum.at[didx.at[b - 1]],
                                                 sems, add=True)
            gd[SEG_G - 1].wait()
            sd[SEG_G - 1] = pltpu.async_copy(rows.at[(SEG_G - 1) % 2],
                                             accum.at[didx.at[SEG_G - 1]],
                                             sems, add=True)
            sd[SEG_G - 2].wait()
            sd[SEG_G - 1].wait()
            return carry

        lax.fori_loop(0, n_groups, group, 0)
        plsc.subcore_barrier()
        pltpu.sync_copy(accum.at[pl.ds(base, rows_per_tile)],
                        out_h.at[cid, pl.ds(base, rows_per_tile)])

    return pl.kernel(
        body,
        out_type=jax.ShapeDtypeStruct((NC, n_nodes_pad, 128), F32),
        mesh=_mesh(),
        scratch_types=[
            pltpu.VMEM((SEG_G, SUB), jnp.int32),
            pltpu.VMEM((SEG_G, SUB), jnp.int32),
            pltpu.VMEM((2, SUB, 128), F32),
            pltpu.VMEM_SHARED((n_nodes_pad, 128), F32),
            pltpu.SemaphoreType.DMA,
            pltpu.SemaphoreType.DMA,
        ],
    )(table, src4, dst4)


def _pool(table, idx4, n_rows_pad, n_perms, k_slots):
    n_chunks = n_rows_pad // CH
    per_worker = -(-n_chunks // NW)
    P, K = n_perms, k_slots

    def body(table_h, idx_h, out_h, ibuf, zbuf, sem):
        cid = lax.axis_index("c")
        sid = lax.axis_index("s")
        wid = sid * NC + cid

        def chunk(t, carry):
            ci = t * NW + wid

            @pl.when(ci < n_chunks)
            def _():
                descs = []
                for p in range(P):
                    for k in range(K):
                        descs.append(pltpu.async_copy(idx_h.at[p, k, ci],
                                                      ibuf.at[p, k], sem))
                for d in descs:
                    d.wait()
                for k in range(K):
                    descs = []
                    for p in range(P):
                        for j in range(2):
                            descs.append(pltpu.async_copy(
                                table_h.at[ibuf.at[p, k, j]],
                                zbuf.at[p, pl.ds(SUB * j, SUB)],
                                sem, add=(k > 0)))
                    for d in descs:
                        d.wait()
                descs = []
                for p in range(P):
                    descs.append(pltpu.async_copy(
                        zbuf.at[p], out_h.at[p, pl.ds(ci * CH, CH)], sem))
                for d in descs:
                    d.wait()

            return carry

        lax.fori_loop(0, per_worker, chunk, 0)

    return pl.kernel(
        body,
        out_type=jax.ShapeDtypeStruct((P, n_rows_pad, 128), jnp.float32),
        mesh=_mesh(),
        scratch_types=[
            pltpu.VMEM((P, K, 2, SUB), jnp.int32),
            pltpu.VMEM((P, CH, 128), jnp.float32),
            pltpu.SemaphoreType.DMA,
        ],
    )(table, idx4)


def _tc_layer(xin, aggs, wself, wneigh, bias):
    n = xin.shape[0]
    blk = 1000

    def kfn(x_ref, a_ref, ws_ref, wn_ref, b_ref, o_ref):
        agg = a_ref[0] + a_ref[1]
        acc = jnp.dot(x_ref[...], ws_ref[...], preferred_element_type=F32)
        acc = acc + jnp.dot(agg, wn_ref[...], preferred_element_type=F32)
        o_ref[...] = jnp.maximum(acc + b_ref[...], 0.0)

    return pl.pallas_call(
        kfn,
        grid=(n // blk,),
        in_specs=[
            pl.BlockSpec((blk, 128), lambda i: (i, 0)),
            pl.BlockSpec((NC, blk, 128), lambda i: (0, i, 0)),
            pl.BlockSpec((128, 128), lambda i: (0, 0)),
            pl.BlockSpec((128, 128), lambda i: (0, 0)),
            pl.BlockSpec((1, 128), lambda i: (0, 0)),
        ],
        out_specs=pl.BlockSpec((blk, 128), lambda i: (i, 0)),
        out_shape=jax.ShapeDtypeStruct((n, 128), F32),
    )(xin, aggs, wself, wneigh, bias.reshape(1, 128))


def _tc_heads(h, aggs, ws2, wn2, b2, wa1, ba1, wa2, ba2, wb1, wg1, wp1, wi1):
    n = h.shape[0]
    blk = 1000

    def kfn(h_ref, a_ref, ws_ref, wn_ref, b_ref, wa1_ref, ba1_ref, wa2_ref,
            ba2_ref, wb1_ref, wg1_ref, wp1_ref, wi1_ref,
            at_ref, tb_ref, tg_ref, tp_ref, ti_ref):
        agg = a_ref[0] + a_ref[1]
        nr = jnp.dot(h_ref[...], ws_ref[...], preferred_element_type=F32)
        nr = nr + jnp.dot(agg, wn_ref[...], preferred_element_type=F32)
        nr = jnp.maximum(nr + b_ref[...], 0.0)
        t = jnp.maximum(jnp.dot(nr, wa1_ref[...], preferred_element_type=F32)
                        + ba1_ref[...], 0.0)
        at_ref[...] = jnp.dot(t, wa2_ref[...], preferred_element_type=F32) + ba2_ref[...]
        for j in range(2):
            tb_ref[j] = jnp.dot(nr, wb1_ref[pl.ds(128 * j, 128)],
                                preferred_element_type=F32)
        for j in range(3):
            tg_ref[j] = jnp.dot(nr, wg1_ref[pl.ds(128 * j, 128)],
                                preferred_element_type=F32)
        for j in range(4):
            tp_ref[j] = jnp.dot(nr, wp1_ref[pl.ds(128 * j, 128)],
                                preferred_element_type=F32)
        for j in range(4):
            ti_ref[j] = jnp.dot(nr, wi1_ref[pl.ds(128 * j, 128)],
                                preferred_element_type=F32)

    full = lambda shape: pl.BlockSpec(shape, lambda i: tuple(0 for _ in shape))
    return pl.pallas_call(
        kfn,
        grid=(n // blk,),
        in_specs=[
            pl.BlockSpec((blk, 128), lambda i: (i, 0)),
            pl.BlockSpec((NC, blk, 128), lambda i: (0, i, 0)),
            full((128, 128)), full((128, 128)), full((1, 128)),
            full((128, 128)), full((1, 128)), full((128, 2)), full((1, 2)),
            full((256, 128)), full((384, 128)), full((512, 128)), full((512, 128)),
        ],
        out_specs=[
            pl.BlockSpec((blk, 2), lambda i: (i, 0)),
            pl.BlockSpec((2, blk, 128), lambda i: (0, i, 0)),
            pl.BlockSpec((3, blk, 128), lambda i: (0, i, 0)),
            pl.BlockSpec((4, blk, 128), lambda i: (0, i, 0)),
            pl.BlockSpec((4, blk, 128), lambda i: (0, i, 0)),
        ],
        out_shape=[
            jax.ShapeDtypeStruct((n, 2), F32),
            jax.ShapeDtypeStruct((2, n, 128), F32),
            jax.ShapeDtypeStruct((3, n, 128), F32),
            jax.ShapeDtypeStruct((4, n, 128), F32),
            jax.ShapeDtypeStruct((4, n, 128), F32),
        ],
    )(h, aggs, ws2, wn2, b2.reshape(1, 128), wa1, ba1.reshape(1, 128), wa2,
      ba2.reshape(1, 2), wb1, wg1, wp1, wi1)


def _tc_readout(z, w2, b1v, b2v, n_rows):
    P, rp, _ = z.shape
    out_d = w2.shape[1]
    blk = 512

    def kfn(z_ref, w2_ref, b1_ref, b2_ref, o_ref):
        acc = None
        for p in range(P):
            y = jnp.maximum(z_ref[p] + b1_ref[...], 0.0)
            yy = jnp.dot(y, w2_ref[...], preferred_element_type=F32)
            acc = yy if acc is None else acc + yy
        o_ref[...] = acc + float(P) * b2_ref[...]

    out = pl.pallas_call(
        kfn,
        grid=(rp // blk,),
        in_specs=[
            pl.BlockSpec((P, blk, 128), lambda i: (0, i, 0)),
            pl.BlockSpec((128, out_d), lambda i: (0, 0)),
            pl.BlockSpec((1, 128), lambda i: (0, 0)),
            pl.BlockSpec((1, out_d), lambda i: (0, 0)),
        ],
        out_specs=pl.BlockSpec((blk, out_d), lambda i: (i, 0)),
        out_shape=jax.ShapeDtypeStruct((rp, out_d), F32),
    )(z, w2, b1v.reshape(1, 128), b2v.reshape(1, out_d))
    return out[:n_rows]


def _readout(table, inter, perms, n_nodes, w2, b1v, b2v):
    r, _ = inter.shape
    P = len(perms)
    K = len(perms[0])
    rp = _ceil_to(r, 512)
    cols = []
    for perm in perms:
        cols.append(jnp.stack([inter[:, perm[j]] + j * n_nodes for j in range(K)]))
    idx = jnp.stack(cols)
    idx = jnp.pad(idx, ((0, 0), (0, 0), (0, rp - r)))
    idx4 = idx.reshape(P, K, rp // CH, 2, SUB)
    z = _pool(table, idx4, rp, P, K)
    return _tc_readout(z, w2, b1v, b2v, r)


def kernel(x, edge_index, bonds, angles, propers, impropers, Ws1, Wn1, b1,
           Ws2, Wn2, b2, Wa1, ba1, Wa2, ba2, Wb1, bb1, Wb2, bb2, Wg1, bg1,
           Wg2, bg2, Wp1, bp1, Wp2, bp2, Wi1, bi1, Wi2, bi2):
    n = x.shape[0]
    e = edge_index.shape[1]
    n_pad = _ceil_to(n + 1, NS * 8)
    e_pad = _ceil_to(e, SEG_G * SUB * NW)

    src = jnp.pad(edge_index[0], (0, e_pad - e))
    dst = jnp.pad(edge_index[1], (0, e_pad - e), constant_values=n)
    n_groups = e_pad // (SEG_G * SUB * NW)
    src4 = src.reshape(n_groups, NW, SEG_G, SUB)
    dst4 = dst.reshape(n_groups, NW, SEG_G, SUB)

    agg1 = _segsum(x, src4, dst4, n_pad)[:, :n]
    h = _tc_layer(x, agg1, Ws1, Wn1, b1)
    agg2 = _segsum(h, src4, dst4, n_pad)[:, :n]
    atoms, tb, tg, tp, ti = _tc_heads(h, agg2, Ws2, Wn2, b2, Wa1, ba1, Wa2,
                                      ba2, Wb1, Wg1, Wp1, Wi1)

    bonds_out = _readout(tb.reshape(2 * n, 128), bonds, [(0, 1), (1, 0)],
                         n, Wb2, bb1, bb2)
    angles_out = _readout(tg.reshape(3 * n, 128), angles,
                          [(0, 1, 2), (2, 1, 0)], n, Wg2, bg1, bg2)
    propers_out = _readout(tp.reshape(4 * n, 128), propers,
                           [(0, 1, 2, 3), (3, 2, 1, 0)], n, Wp2, bp1, bp2)
    imp_perms = [(0, 1, 2, 3), (2, 1, 3, 0), (3, 1, 0, 2)]
    impropers_out = _readout(ti.reshape(4 * n, 128), impropers, imp_perms,
                             n, Wi2, bi1, bi2)
    return (atoms, bonds_out, angles_out, propers_out, impropers_out)
